# baseline (speedup 1.0000x reference)
import dataclasses
import functools
import jax
import jax.numpy as jnp
from jax import lax
from jax.experimental import pallas as pl
from jax.experimental.pallas import tpu as pltpu
from jax.experimental.pallas import tpu_sc as plsc

N = 10000
D = 128
E = 320000
NC = 2
NS = 16
NW = NC * NS
EW = E // NW
CH = 128
NK = -(-EW // CH)
EWP = NK * CH
EPAD = NW * EWP
NPAD = 10240
SH = NPAD // NS
R = 400


def _tc_project(x, W, a_s, a_d):
    def body(x_ref, w_ref, as_ref, ad_ref, xs_ref, als_ref, ald_ref):
        xs = jnp.dot(x_ref[...], w_ref[...], preferred_element_type=jnp.float32)
        xs_ref[...] = xs
        als_ref[...] = jnp.dot(xs, as_ref[...], preferred_element_type=jnp.float32)
        ald_ref[...] = jnp.dot(xs, ad_ref[...], preferred_element_type=jnp.float32)
    return pl.pallas_call(
        body,
        grid=(N // R,),
        in_specs=[pl.BlockSpec((R, D), lambda i: (i, 0)),
                  pl.BlockSpec((D, D), lambda i: (0, 0)),
                  pl.BlockSpec((D, 1), lambda i: (0, 0)),
                  pl.BlockSpec((D, 1), lambda i: (0, 0))],
        out_specs=[pl.BlockSpec((R, D), lambda i: (i, 0)),
                   pl.BlockSpec((R, 1), lambda i: (i, 0)),
                   pl.BlockSpec((R, 1), lambda i: (i, 0))],
        out_shape=[jax.ShapeDtypeStruct((N, D), jnp.float32),
                   jax.ShapeDtypeStruct((N, 1), jnp.float32),
                   jax.ShapeDtypeStruct((N, 1), jnp.float32)],
    )(x, W, a_s.reshape(D, 1), a_d.reshape(D, 1))


def _tc_combine_project(acc, den, b, W, a_s, a_d):
    def body(acc_ref, den_ref, b_ref, w_ref, as_ref, ad_ref,
             xs_ref, als_ref, ald_ref):
        comb = acc_ref[0] + acc_ref[1]
        dd = den_ref[0] + den_ref[1] + 1e-16
        h = jnp.maximum(comb / dd + b_ref[...], 0.0)
        xs = jnp.dot(h, w_ref[...], preferred_element_type=jnp.float32)
        xs_ref[...] = xs
        als_ref[...] = jnp.dot(xs, as_ref[...], preferred_element_type=jnp.float32)
        ald_ref[...] = jnp.dot(xs, ad_ref[...], preferred_element_type=jnp.float32)
    return pl.pallas_call(
        body,
        grid=(N // R,),
        in_specs=[pl.BlockSpec((2, R, D), lambda i: (0, i, 0)),
                  pl.BlockSpec((2, R, 1), lambda i: (0, i, 0)),
                  pl.BlockSpec((1, D), lambda i: (0, 0)),
                  pl.BlockSpec((D, D), lambda i: (0, 0)),
                  pl.BlockSpec((D, 1), lambda i: (0, 0)),
                  pl.BlockSpec((D, 1), lambda i: (0, 0))],
        out_specs=[pl.BlockSpec((R, D), lambda i: (i, 0)),
                   pl.BlockSpec((R, 1), lambda i: (i, 0)),
                   pl.BlockSpec((R, 1), lambda i: (i, 0))],
        out_shape=[jax.ShapeDtypeStruct((N, D), jnp.float32),
                   jax.ShapeDtypeStruct((N, 1), jnp.float32),
                   jax.ShapeDtypeStruct((N, 1), jnp.float32)],
    )(acc, den.reshape(NC, NPAD, 1), b.reshape(1, D),
      W, a_s.reshape(D, 1), a_d.reshape(D, 1))


def _tc_combine_final(acc, den, b):
    def body(acc_ref, den_ref, b_ref, out_ref):
        comb = acc_ref[0] + acc_ref[1]
        dd = den_ref[0] + den_ref[1] + 1e-16
        out_ref[...] = comb / dd + b_ref[...]
    return pl.pallas_call(
        body,
        grid=(N // R,),
        in_specs=[pl.BlockSpec((2, R, D), lambda i: (0, i, 0)),
                  pl.BlockSpec((2, R, 1), lambda i: (0, i, 0)),
                  pl.BlockSpec((1, D), lambda i: (0, 0))],
        out_specs=pl.BlockSpec((R, D), lambda i: (i, 0)),
        out_shape=jax.ShapeDtypeStruct((N, D), jnp.float32),
    )(acc, den.reshape(NC, NPAD, 1), b.reshape(1, D))


def _sc_edge_layer(xs, asrc, adst, src3, dst3):
    mesh = plsc.VectorSubcoreMesh(core_axis_name="c", subcore_axis_name="s")
    cp = pltpu.CompilerParams()
    if "needs_layout_passes" in pltpu.CompilerParams.__dataclass_fields__:
        cp = dataclasses.replace(cp, needs_layout_passes=False)

    @functools.partial(
        pl.kernel,
        compiler_params=cp,
        out_type=[jax.ShapeDtypeStruct((NC, NPAD, D), jnp.float32),
                  jax.ShapeDtypeStruct((NC, NPAD), jnp.float32)],
        mesh=mesh,
        scratch_types=[
            pltpu.VMEM((1, CH), jnp.int32),
            pltpu.VMEM((1, CH), jnp.int32),
            pltpu.VMEM((1, CH), jnp.int32),
            pltpu.VMEM((1, CH), jnp.int32),
            pltpu.VMEM((CH,), jnp.float32),
            pltpu.VMEM((CH,), jnp.float32),
            pltpu.VMEM((CH,), jnp.float32),
            pltpu.VMEM((CH,), jnp.float32),
            pltpu.VMEM((CH,), jnp.float32),
            pltpu.VMEM((CH,), jnp.float32),
            pltpu.VMEM((CH, D), jnp.float32),
            pltpu.VMEM((CH, D), jnp.float32),
            pltpu.VMEM((SH,), jnp.float32),
            pltpu.VMEM_SHARED((NPAD, D), jnp.float32),
            pltpu.VMEM_SHARED((NPAD,), jnp.float32),
            pltpu.SemaphoreType.DMA,
            pltpu.SemaphoreType.DMA,
            pltpu.SemaphoreType.DMA,
            pltpu.SemaphoreType.DMA,
            pltpu.SemaphoreType.DMA,
            pltpu.SemaphoreType.DMA,
        ],
    )
    def k(xs_hbm, asrc_hbm, adst_hbm, src_hbm, dst_hbm,
          acc_hbm, den_hbm,
          src0, src1, dst0, dst1, a10, a11, a20, a21, ex0, ex1,
          rows0, rows1, zden_v, acc_sh, den_sh,
          isem0, isem1, gsem0, gsem1, ssem0, ssem1):
        cid = lax.axis_index("c")
        sid = lax.axis_index("s")
        wid = cid * NS + sid
        z16 = jnp.zeros((16,), jnp.float32)
        srcb, dstb = (src0, src1), (dst0, dst1)
        a1b, a2b = (a10, a11), (a20, a21)
        exb, rowsb = (ex0, ex1), (rows0, rows1)
        isem, gsem, ssem = (isem0, isem1), (gsem0, gsem1), (ssem0, ssem1)

        @pl.loop(0, CH)
        def _(r):
            for c in range(D // 16):
                rows0[r, pl.ds(c * 16, 16)] = z16

        @pl.loop(0, SH, step=16)
        def _(i):
            zden_v[pl.ds(i, 16)] = z16

        for i in range(SH // CH):
            pltpu.sync_copy(rows0, acc_sh.at[pl.ds(sid * SH + i * CH, CH)])
        pltpu.sync_copy(zden_v, den_sh.at[pl.ds(sid * SH, SH)])

        plsc.subcore_barrier()

        def fire_idx(b, kk):
            pltpu.async_copy(src_hbm.at[wid, pl.ds(kk, 1)], srcb[b], isem[b])
            pltpu.async_copy(dst_hbm.at[wid, pl.ds(kk, 1)], dstb[b], isem[b])

        def wait_idx(b, kk):
            pltpu.make_async_copy(src_hbm.at[wid, pl.ds(kk, 1)], srcb[b], isem[b]).wait()
            pltpu.make_async_copy(dst_hbm.at[wid, pl.ds(kk, 1)], dstb[b], isem[b]).wait()

        def fire_gathers(b):
            pltpu.async_copy(asrc_hbm.at[srcb[b].at[0]], a1b[b], gsem[b])
            pltpu.async_copy(adst_hbm.at[dstb[b].at[0]], a2b[b], gsem[b])
            pltpu.async_copy(xs_hbm.at[srcb[b].at[0]], rowsb[b], gsem[b])

        def wait_gathers(b):
            pltpu.make_async_copy(asrc_hbm.at[srcb[b].at[0]], a1b[b], gsem[b]).wait()
            pltpu.make_async_copy(adst_hbm.at[dstb[b].at[0]], a2b[b], gsem[b]).wait()
            pltpu.make_async_copy(xs_hbm.at[srcb[b].at[0]], rowsb[b], gsem[b]).wait()

        def fire_scatters(b):
            pltpu.async_copy(exb[b], den_sh.at[dstb[b].at[0]], ssem[b], add=True)
            pltpu.async_copy(rowsb[b], acc_sh.at[dstb[b].at[0]], ssem[b], add=True)

        def wait_scatters(b):
            pltpu.make_async_copy(exb[b], den_sh.at[dstb[b].at[0]], ssem[b]).wait()
            pltpu.make_async_copy(rowsb[b], acc_sh.at[dstb[b].at[0]], ssem[b]).wait()

        def compute_ex(b, kk):
            base = wid * EWP + kk * CH
            lanes = lax.iota(jnp.int32, 16)

            @pl.loop(0, CH, step=16)
            def _(j):
                e = a1b[b][pl.ds(j, 16)] + a2b[b][pl.ds(j, 16)]
                e = jnp.maximum(e, 0.2 * e)
                ok = (base + j + lanes) < E
                exb[b][pl.ds(j, 16)] = jnp.where(ok, jnp.exp(e), 0.0)

        def scale_rows(b):
            @pl.loop(0, CH, step=16)
            def _(j):
                ex16 = exb[b][pl.ds(j, 16)]
                for rr in range(16):
                    sv = ex16[rr]
                    for c in range(D // 16):
                        sl = pl.ds(c * 16, 16)
                        rowsb[b][j + rr, sl] = rowsb[b][j + rr, sl] * sv

        fire_idx(0, 0)
        wait_idx(0, 0)
        fire_gathers(0)

        @pl.loop(0, NK - 1, step=2)
        def _(kk):
            @pl.when(kk > 0)
            def _():
                wait_scatters(1)
            fire_idx(1, kk + 1)
            wait_gathers(0)
            compute_ex(0, kk)
            wait_idx(1, kk + 1)
            fire_gathers(1)
            scale_rows(0)
            fire_scatters(0)
            wait_gathers(1)
            compute_ex(1, kk + 1)
            wait_scatters(0)
            fire_idx(0, kk + 2)
            wait_idx(0, kk + 2)
            fire_gathers(0)
            scale_rows(1)
            fire_scatters(1)

        wait_gathers(0)
        compute_ex(0, NK - 1)
        scale_rows(0)
        wait_scatters(1)
        pltpu.sync_copy(ex0, den_sh.at[dst0.at[0]], add=True)
        pltpu.sync_copy(rows0, acc_sh.at[dst0.at[0]], add=True)

        plsc.subcore_barrier()

        pltpu.sync_copy(acc_sh.at[pl.ds(sid * SH, SH)],
                        acc_hbm.at[cid, pl.ds(sid * SH, SH)])
        pltpu.sync_copy(den_sh.at[pl.ds(sid * SH, SH)],
                        den_hbm.at[cid, pl.ds(sid * SH, SH)])

    return k(xs, asrc, adst, src3, dst3)


def kernel(x, edge_index, edge_attr, W1, a_s1, a_d1, b1, W2, a_s2, a_d2, b2):
    ei = jnp.pad(edge_index.astype(jnp.int32), ((0, 0), (0, EPAD - E)))
    src3 = ei[0].reshape(NW, NK, CH)
    dst3 = ei[1].reshape(NW, NK, CH)

    xs1, as1, ad1 = _tc_project(x, W1, a_s1, a_d1)
    acc1, den1 = _sc_edge_layer(xs1, as1.reshape(N), ad1.reshape(N), src3, dst3)
    xs2, as2, ad2 = _tc_combine_project(acc1, den1, b1, W2, a_s2, a_d2)
    acc2, den2 = _sc_edge_layer(xs2, as2.reshape(N), ad2.reshape(N), src3, dst3)
    return _tc_combine_final(acc2, den2, b2)


# speedup vs baseline: 1.3121x; 1.3121x over previous
import dataclasses
import functools
import jax
import jax.numpy as jnp
from jax import lax
from jax.experimental import pallas as pl
from jax.experimental.pallas import tpu as pltpu
from jax.experimental.pallas import tpu_sc as plsc

N = 10000
D = 128
E = 320000
NC = 2
NS = 16
NW = NC * NS
EW = E // NW
CH = 80
NK = -(-EW // CH)
EWP = NK * CH
EPAD = NW * EWP
NPAD = 10240
SH = NPAD // NS
NPA = 10112
SHA = NPA // NS
R = 400


def _tc_project(x, W, a_s, a_d):
    def body(x_ref, w_ref, as_ref, ad_ref, xs_ref, als_ref, ald_ref):
        xs = jnp.dot(x_ref[...], w_ref[...], preferred_element_type=jnp.float32)
        xs_ref[...] = xs
        als_ref[...] = jnp.dot(xs, as_ref[...], preferred_element_type=jnp.float32)
        ald_ref[...] = jnp.dot(xs, ad_ref[...], preferred_element_type=jnp.float32)
    return pl.pallas_call(
        body,
        grid=(N // R,),
        in_specs=[pl.BlockSpec((R, D), lambda i: (i, 0)),
                  pl.BlockSpec((D, D), lambda i: (0, 0)),
                  pl.BlockSpec((D, 1), lambda i: (0, 0)),
                  pl.BlockSpec((D, 1), lambda i: (0, 0))],
        out_specs=[pl.BlockSpec((R, D), lambda i: (i, 0)),
                   pl.BlockSpec((R, 1), lambda i: (i, 0)),
                   pl.BlockSpec((R, 1), lambda i: (i, 0))],
        out_shape=[jax.ShapeDtypeStruct((N, D), jnp.float32),
                   jax.ShapeDtypeStruct((N, 1), jnp.float32),
                   jax.ShapeDtypeStruct((N, 1), jnp.float32)],
    )(x, W, a_s.reshape(D, 1), a_d.reshape(D, 1))


def _tc_combine_project(acc, den, b, W, a_s, a_d):
    def body(acc_ref, den_ref, b_ref, w_ref, as_ref, ad_ref,
             xs_ref, als_ref, ald_ref):
        comb = acc_ref[0] + acc_ref[1]
        dd = den_ref[0] + den_ref[1] + 1e-16
        h = jnp.maximum(comb / dd + b_ref[...], 0.0)
        xs = jnp.dot(h, w_ref[...], preferred_element_type=jnp.float32)
        xs_ref[...] = xs
        als_ref[...] = jnp.dot(xs, as_ref[...], preferred_element_type=jnp.float32)
        ald_ref[...] = jnp.dot(xs, ad_ref[...], preferred_element_type=jnp.float32)
    return pl.pallas_call(
        body,
        grid=(N // R,),
        in_specs=[pl.BlockSpec((2, R, D), lambda i: (0, i, 0)),
                  pl.BlockSpec((2, R, 1), lambda i: (0, i, 0)),
                  pl.BlockSpec((1, D), lambda i: (0, 0)),
                  pl.BlockSpec((D, D), lambda i: (0, 0)),
                  pl.BlockSpec((D, 1), lambda i: (0, 0)),
                  pl.BlockSpec((D, 1), lambda i: (0, 0))],
        out_specs=[pl.BlockSpec((R, D), lambda i: (i, 0)),
                   pl.BlockSpec((R, 1), lambda i: (i, 0)),
                   pl.BlockSpec((R, 1), lambda i: (i, 0))],
        out_shape=[jax.ShapeDtypeStruct((N, D), jnp.float32),
                   jax.ShapeDtypeStruct((N, 1), jnp.float32),
                   jax.ShapeDtypeStruct((N, 1), jnp.float32)],
    )(acc, den.reshape(NC, NPAD, 1), b.reshape(1, D),
      W, a_s.reshape(D, 1), a_d.reshape(D, 1))


def _tc_combine_final(acc, den, b):
    def body(acc_ref, den_ref, b_ref, out_ref):
        comb = acc_ref[0] + acc_ref[1]
        dd = den_ref[0] + den_ref[1] + 1e-16
        out_ref[...] = comb / dd + b_ref[...]
    return pl.pallas_call(
        body,
        grid=(N // R,),
        in_specs=[pl.BlockSpec((2, R, D), lambda i: (0, i, 0)),
                  pl.BlockSpec((2, R, 1), lambda i: (0, i, 0)),
                  pl.BlockSpec((1, D), lambda i: (0, 0))],
        out_specs=pl.BlockSpec((R, D), lambda i: (i, 0)),
        out_shape=jax.ShapeDtypeStruct((N, D), jnp.float32),
    )(acc, den.reshape(NC, NPAD, 1), b.reshape(1, D))


def _sc_edge_layer(xs, asrc, adst, src3, dst3):
    mesh = plsc.VectorSubcoreMesh(core_axis_name="c", subcore_axis_name="s")
    cp = pltpu.CompilerParams()
    if "needs_layout_passes" in pltpu.CompilerParams.__dataclass_fields__:
        cp = dataclasses.replace(cp, needs_layout_passes=False)

    @functools.partial(
        pl.kernel,
        compiler_params=cp,
        out_type=[jax.ShapeDtypeStruct((NC, NPA, D), jnp.float32),
                  jax.ShapeDtypeStruct((NC, NPAD), jnp.float32)],
        mesh=mesh,
        scratch_types=[
            pltpu.VMEM((1, CH), jnp.int32),
            pltpu.VMEM((1, CH), jnp.int32),
            pltpu.VMEM((1, CH), jnp.int32),
            pltpu.VMEM((NK, CH), jnp.int32),
            pltpu.VMEM((CH,), jnp.float32),
            pltpu.VMEM((CH,), jnp.float32),
            pltpu.VMEM((CH,), jnp.float32),
            pltpu.VMEM((CH,), jnp.float32),
            pltpu.VMEM((CH,), jnp.float32),
            pltpu.VMEM((CH,), jnp.float32),
            pltpu.VMEM((CH,), jnp.float32),
            pltpu.VMEM((CH,), jnp.float32),
            pltpu.VMEM((CH,), jnp.float32),
            pltpu.VMEM((CH, D), jnp.float32),
            pltpu.VMEM((CH, D), jnp.float32),
            pltpu.VMEM((CH, D), jnp.float32),
            pltpu.VMEM_SHARED((NPA, D), jnp.float32),
            pltpu.VMEM_SHARED((NPAD,), jnp.float32),
            pltpu.SemaphoreType.DMA,
            pltpu.SemaphoreType.DMA,
            pltpu.SemaphoreType.DMA,
            pltpu.SemaphoreType.DMA,
            pltpu.SemaphoreType.DMA,
            pltpu.SemaphoreType.DMA,
            pltpu.SemaphoreType.DMA,
            pltpu.SemaphoreType.DMA,
            pltpu.SemaphoreType.DMA,
        ],
    )
    def k(xs_hbm, asrc_hbm, adst_hbm, src_hbm, dst_hbm,
          acc_hbm, den_hbm,
          src0, src1, src2, dstf, a10, a11, a12, a20, a21, a22,
          ex0, ex1, ex2, rows0, rows1, rows2, acc_sh, den_sh,
          isem0, isem1, isem2, gsem0, gsem1, gsem2, ssem0, ssem1, ssem2):
        cid = lax.axis_index("c")
        sid = lax.axis_index("s")
        wid = cid * NS + sid
        z16 = jnp.zeros((16,), jnp.float32)
        srcb = (src0, src1, src2)
        a1b, a2b = (a10, a11, a12), (a20, a21, a22)
        exb, rowsb = (ex0, ex1, ex2), (rows0, rows1, rows2)
        isem, gsem, ssem = (isem0, isem1, isem2), (gsem0, gsem1, gsem2), (ssem0, ssem1, ssem2)

        @pl.loop(0, CH)
        def _(r):
            for c in range(D // 16):
                rows0[r, pl.ds(c * 16, 16)] = z16

        @pl.loop(0, CH, step=16)
        def _(i):
            ex0[pl.ds(i, 16)] = z16

        for i in range(SHA // CH):
            pltpu.sync_copy(rows0, acc_sh.at[pl.ds(sid * SHA + i * CH, CH)])
        pltpu.sync_copy(rows0.at[pl.ds(0, SHA % CH)],
                        acc_sh.at[pl.ds(sid * SHA + (SHA // CH) * CH, SHA % CH)])
        for i in range(SH // CH):
            pltpu.sync_copy(ex0, den_sh.at[pl.ds(sid * SH + i * CH, CH)])

        pltpu.sync_copy(dst_hbm.at[wid], dstf)

        plsc.subcore_barrier()

        def fire_idx(b, kk):
            pltpu.async_copy(src_hbm.at[wid, pl.ds(kk, 1)], srcb[b], isem[b])

        def wait_idx(b, kk):
            pltpu.make_async_copy(src_hbm.at[wid, pl.ds(kk, 1)], srcb[b], isem[b]).wait()

        def fire_gathers(b, kk):
            pltpu.async_copy(asrc_hbm.at[srcb[b].at[0]], a1b[b], gsem[b])
            pltpu.async_copy(adst_hbm.at[dstf.at[kk]], a2b[b], gsem[b])
            pltpu.async_copy(xs_hbm.at[srcb[b].at[0]], rowsb[b], gsem[b])

        def wait_gathers(b, kk):
            pltpu.make_async_copy(asrc_hbm.at[srcb[b].at[0]], a1b[b], gsem[b]).wait()
            pltpu.make_async_copy(adst_hbm.at[dstf.at[kk]], a2b[b], gsem[b]).wait()
            pltpu.make_async_copy(xs_hbm.at[srcb[b].at[0]], rowsb[b], gsem[b]).wait()

        def fire_scatters(b, kk):
            pltpu.async_copy(exb[b], den_sh.at[dstf.at[kk]], ssem[b], add=True)
            pltpu.async_copy(rowsb[b], acc_sh.at[dstf.at[kk]], ssem[b], add=True)

        def wait_scatters(b, kk):
            pltpu.make_async_copy(exb[b], den_sh.at[dstf.at[kk]], ssem[b]).wait()
            pltpu.make_async_copy(rowsb[b], acc_sh.at[dstf.at[kk]], ssem[b]).wait()

        def compute_ex(b, kk):
            if EPAD == E:
                @pl.loop(0, CH, step=16)
                def _(j):
                    e = a1b[b][pl.ds(j, 16)] + a2b[b][pl.ds(j, 16)]
                    e = jnp.maximum(e, 0.2 * e)
                    exb[b][pl.ds(j, 16)] = jnp.exp(e)
            else:
                base = wid * EWP + kk * CH
                lanes = lax.iota(jnp.int32, 16)

                @pl.loop(0, CH, step=16)
                def _(j):
                    e = a1b[b][pl.ds(j, 16)] + a2b[b][pl.ds(j, 16)]
                    e = jnp.maximum(e, 0.2 * e)
                    ok = (base + j + lanes) < E
                    exb[b][pl.ds(j, 16)] = jnp.where(ok, jnp.exp(e), 0.0)

        def scale_rows(b):
            @pl.loop(0, CH, step=16)
            def _(j):
                ex16 = exb[b][pl.ds(j, 16)]
                for rr in range(16):
                    sv = ex16[rr]
                    for c in range(D // 16):
                        sl = pl.ds(c * 16, 16)
                        rowsb[b][j + rr, sl] = rowsb[b][j + rr, sl] * sv

        def process(c, i, wait_sc=True, fire_next=True):
            wait_gathers(i, c)
            compute_ex(i, c)
            if wait_sc:
                wait_scatters((i + 1) % 3, c - 2)
            if fire_next:
                j = (i + 1) % 3
                fire_idx(j, c + 1)
                wait_idx(j, c + 1)
                fire_gathers(j, c + 1)
            scale_rows(i)
            fire_scatters(i, c)

        fire_idx(0, 0)
        wait_idx(0, 0)
        fire_gathers(0, 0)
        process(0, 0, wait_sc=False)
        process(1, 1, wait_sc=False)
        process(2, 2)

        @pl.loop(1, (NK - 2) // 3)
        def _(t):
            c = 3 * t
            process(c, 0)
            process(c + 1, 1)
            process(c + 2, 2)

        process(NK - 2, (NK - 2) % 3)
        process(NK - 1, (NK - 1) % 3, fire_next=False)
        wait_scatters((NK - 2) % 3, NK - 2)
        wait_scatters((NK - 1) % 3, NK - 1)

        plsc.subcore_barrier()

        pltpu.sync_copy(acc_sh.at[pl.ds(sid * SHA, SHA)],
                        acc_hbm.at[cid, pl.ds(sid * SHA, SHA)])
        pltpu.sync_copy(den_sh.at[pl.ds(sid * SH, SH)],
                        den_hbm.at[cid, pl.ds(sid * SH, SH)])

    return k(xs, asrc, adst, src3, dst3)


def kernel(x, edge_index, edge_attr, W1, a_s1, a_d1, b1, W2, a_s2, a_d2, b2):
    ei = jnp.pad(edge_index.astype(jnp.int32), ((0, 0), (0, EPAD - E)))
    src3 = ei[0].reshape(NW, NK, CH)
    dst3 = ei[1].reshape(NW, NK, CH)

    xs1, as1, ad1 = _tc_project(x, W1, a_s1, a_d1)
    acc1, den1 = _sc_edge_layer(xs1, as1.reshape(N), ad1.reshape(N), src3, dst3)
    xs2, as2, ad2 = _tc_combine_project(acc1, den1, b1, W2, a_s2, a_d2)
    acc2, den2 = _sc_edge_layer(xs2, as2.reshape(N), ad2.reshape(N), src3, dst3)
    return _tc_combine_final(acc2, den2, b2)


# speedup vs baseline: 1.4811x; 1.1288x over previous
import dataclasses
import functools
import jax
import jax.numpy as jnp
from jax import lax
from jax.experimental import pallas as pl
from jax.experimental.pallas import tpu as pltpu
from jax.experimental.pallas import tpu_sc as plsc

N = 10000
D = 128
E = 320000
NC = 2
NS = 16
NW = NC * NS
EW = E // NW
CH = 80
NK = -(-EW // CH)
EWP = NK * CH
EPAD = NW * EWP
NPAD = 10240
SH = NPAD // NS
R = 2000


def _tc_project(x, W, a_s, a_d):
    def body(x_ref, w_ref, as_ref, ad_ref, xs_ref, als_ref, ald_ref):
        xs = jnp.dot(x_ref[...], w_ref[...], preferred_element_type=jnp.float32)
        xs_ref[...] = xs
        als_ref[...] = jnp.dot(xs, as_ref[...], preferred_element_type=jnp.float32)
        ald_ref[...] = jnp.dot(xs, ad_ref[...], preferred_element_type=jnp.float32)
    return pl.pallas_call(
        body,
        grid=(N // R,),
        in_specs=[pl.BlockSpec((R, D), lambda i: (i, 0)),
                  pl.BlockSpec((D, D), lambda i: (0, 0)),
                  pl.BlockSpec((D, 1), lambda i: (0, 0)),
                  pl.BlockSpec((D, 1), lambda i: (0, 0))],
        out_specs=[pl.BlockSpec((R, D), lambda i: (i, 0)),
                   pl.BlockSpec((R, 1), lambda i: (i, 0)),
                   pl.BlockSpec((R, 1), lambda i: (i, 0))],
        out_shape=[jax.ShapeDtypeStruct((N, D), jnp.float32),
                   jax.ShapeDtypeStruct((N, 1), jnp.float32),
                   jax.ShapeDtypeStruct((N, 1), jnp.float32)],
    )(x, W, a_s.reshape(D, 1), a_d.reshape(D, 1))


def _tc_combine_project(acc, den, b, W, a_s, a_d):
    def body(acc_ref, den_ref, b_ref, w_ref, as_ref, ad_ref,
             xs_ref, als_ref, ald_ref):
        comb = acc_ref[0] + acc_ref[1]
        dd = den_ref[0] + den_ref[1] + 1e-16
        h = jnp.maximum(comb / dd + b_ref[...], 0.0)
        xs = jnp.dot(h, w_ref[...], preferred_element_type=jnp.float32)
        xs_ref[...] = xs
        als_ref[...] = jnp.dot(xs, as_ref[...], preferred_element_type=jnp.float32)
        ald_ref[...] = jnp.dot(xs, ad_ref[...], preferred_element_type=jnp.float32)
    return pl.pallas_call(
        body,
        grid=(N // R,),
        in_specs=[pl.BlockSpec((2, R, D), lambda i: (0, i, 0)),
                  pl.BlockSpec((2, R, 1), lambda i: (0, i, 0)),
                  pl.BlockSpec((1, D), lambda i: (0, 0)),
                  pl.BlockSpec((D, D), lambda i: (0, 0)),
                  pl.BlockSpec((D, 1), lambda i: (0, 0)),
                  pl.BlockSpec((D, 1), lambda i: (0, 0))],
        out_specs=[pl.BlockSpec((R, D), lambda i: (i, 0)),
                   pl.BlockSpec((R, 1), lambda i: (i, 0)),
                   pl.BlockSpec((R, 1), lambda i: (i, 0))],
        out_shape=[jax.ShapeDtypeStruct((N, D), jnp.float32),
                   jax.ShapeDtypeStruct((N, 1), jnp.float32),
                   jax.ShapeDtypeStruct((N, 1), jnp.float32)],
    )(acc, den.reshape(NC, NPAD, 1), b.reshape(1, D),
      W, a_s.reshape(D, 1), a_d.reshape(D, 1))


def _tc_combine_final(acc, den, b):
    def body(acc_ref, den_ref, b_ref, out_ref):
        comb = acc_ref[0] + acc_ref[1]
        dd = den_ref[0] + den_ref[1] + 1e-16
        out_ref[...] = comb / dd + b_ref[...]
    return pl.pallas_call(
        body,
        grid=(N // R,),
        in_specs=[pl.BlockSpec((2, R, D), lambda i: (0, i, 0)),
                  pl.BlockSpec((2, R, 1), lambda i: (0, i, 0)),
                  pl.BlockSpec((1, D), lambda i: (0, 0))],
        out_specs=pl.BlockSpec((R, D), lambda i: (i, 0)),
        out_shape=jax.ShapeDtypeStruct((N, D), jnp.float32),
    )(acc, den.reshape(NC, NPAD, 1), b.reshape(1, D))


def _sc_edge_layer(xs, asrc, adst, sd3):
    mesh = plsc.VectorSubcoreMesh(core_axis_name="c", subcore_axis_name="s")
    cp = pltpu.CompilerParams()
    if "needs_layout_passes" in pltpu.CompilerParams.__dataclass_fields__:
        cp = dataclasses.replace(cp, needs_layout_passes=False)

    @functools.partial(
        pl.kernel,
        compiler_params=cp,
        out_type=[jax.ShapeDtypeStruct((NC, NPAD, D), jnp.float32),
                  jax.ShapeDtypeStruct((NC, NPAD), jnp.float32)],
        mesh=mesh,
        scratch_types=[
            pltpu.VMEM((2, CH), jnp.int32),
            pltpu.VMEM((2, CH), jnp.int32),
            pltpu.VMEM((CH,), jnp.float32),
            pltpu.VMEM((CH,), jnp.float32),
            pltpu.VMEM((CH,), jnp.float32),
            pltpu.VMEM((CH,), jnp.float32),
            pltpu.VMEM((CH,), jnp.float32),
            pltpu.VMEM((CH,), jnp.float32),
            pltpu.VMEM((CH, D), jnp.float32),
            pltpu.VMEM((CH, D), jnp.float32),
            pltpu.VMEM((SH,), jnp.float32),
            pltpu.VMEM_SHARED((NPAD, D), jnp.float32),
            pltpu.VMEM_SHARED((NPAD,), jnp.float32),
            pltpu.SemaphoreType.DMA,
            pltpu.SemaphoreType.DMA,
            pltpu.SemaphoreType.DMA,
            pltpu.SemaphoreType.DMA,
            pltpu.SemaphoreType.DMA,
            pltpu.SemaphoreType.DMA,
        ],
    )
    def k(xs_hbm, asrc_hbm, adst_hbm, sd_hbm,
          acc_hbm, den_hbm,
          sd0, sd1, a10, a11, a20, a21, ex0, ex1,
          rows0, rows1, zden_v, acc_sh, den_sh,
          isem0, isem1, gsem0, gsem1, ssem0, ssem1):
        cid = lax.axis_index("c")
        sid = lax.axis_index("s")
        wid = cid * NS + sid
        z16 = jnp.zeros((16,), jnp.float32)
        sdb = (sd0, sd1)
        a1b, a2b = (a10, a11), (a20, a21)
        exb, rowsb = (ex0, ex1), (rows0, rows1)
        isem, gsem, ssem = (isem0, isem1), (gsem0, gsem1), (ssem0, ssem1)

        @pl.loop(0, CH)
        def _(r):
            for c in range(D // 16):
                rows0[r, pl.ds(c * 16, 16)] = z16

        @pl.loop(0, SH, step=16)
        def _(i):
            zden_v[pl.ds(i, 16)] = z16

        for i in range(SH // CH):
            pltpu.sync_copy(rows0, acc_sh.at[pl.ds(sid * SH + i * CH, CH)])
        pltpu.sync_copy(zden_v, den_sh.at[pl.ds(sid * SH, SH)])

        plsc.subcore_barrier()

        def fire_idx(b, kk):
            pltpu.async_copy(sd_hbm.at[wid, kk], sdb[b], isem[b])

        def wait_idx(b, kk):
            pltpu.make_async_copy(sd_hbm.at[wid, kk], sdb[b], isem[b]).wait()

        def fire_gathers(b):
            pltpu.async_copy(asrc_hbm.at[sdb[b].at[0]], a1b[b], gsem[b])
            pltpu.async_copy(adst_hbm.at[sdb[b].at[1]], a2b[b], gsem[b])
            pltpu.async_copy(xs_hbm.at[sdb[b].at[0]], rowsb[b], gsem[b])

        def wait_gathers(b):
            pltpu.make_async_copy(asrc_hbm.at[sdb[b].at[0]], a1b[b], gsem[b]).wait()
            pltpu.make_async_copy(adst_hbm.at[sdb[b].at[1]], a2b[b], gsem[b]).wait()
            pltpu.make_async_copy(xs_hbm.at[sdb[b].at[0]], rowsb[b], gsem[b]).wait()

        def fire_scatters(b):
            pltpu.async_copy(exb[b], den_sh.at[sdb[b].at[1]], ssem[b], add=True)
            pltpu.async_copy(rowsb[b], acc_sh.at[sdb[b].at[1]], ssem[b], add=True)

        def wait_scatters(b):
            pltpu.make_async_copy(exb[b], den_sh.at[sdb[b].at[1]], ssem[b]).wait()
            pltpu.make_async_copy(rowsb[b], acc_sh.at[sdb[b].at[1]], ssem[b]).wait()

        def compute_ex(b, kk):
            if EPAD == E:
                @pl.loop(0, CH, step=16)
                def _(j):
                    e = a1b[b][pl.ds(j, 16)] + a2b[b][pl.ds(j, 16)]
                    e = jnp.maximum(e, 0.2 * e)
                    exb[b][pl.ds(j, 16)] = jnp.exp(e)
                return
            base = wid * EWP + kk * CH
            lanes = lax.iota(jnp.int32, 16)

            @pl.loop(0, CH, step=16)
            def _(j):
                e = a1b[b][pl.ds(j, 16)] + a2b[b][pl.ds(j, 16)]
                e = jnp.maximum(e, 0.2 * e)
                ok = (base + j + lanes) < E
                exb[b][pl.ds(j, 16)] = jnp.where(ok, jnp.exp(e), 0.0)

        def scale_rows(b):
            @pl.loop(0, CH, step=16)
            def _(j):
                ex16 = exb[b][pl.ds(j, 16)]
                for rr in range(16):
                    sv = ex16[rr]
                    for c in range(D // 16):
                        sl = pl.ds(c * 16, 16)
                        rowsb[b][j + rr, sl] = rowsb[b][j + rr, sl] * sv

        fire_idx(0, 0)
        wait_idx(0, 0)
        fire_gathers(0)

        @pl.loop(0, NK - 1, step=2)
        def _(kk):
            @pl.when(kk > 0)
            def _():
                wait_scatters(1)
            fire_idx(1, kk + 1)
            wait_gathers(0)
            compute_ex(0, kk)
            wait_idx(1, kk + 1)
            fire_gathers(1)
            scale_rows(0)
            fire_scatters(0)
            wait_gathers(1)
            compute_ex(1, kk + 1)
            wait_scatters(0)
            fire_idx(0, kk + 2)
            wait_idx(0, kk + 2)
            fire_gathers(0)
            scale_rows(1)
            fire_scatters(1)

        wait_gathers(0)
        compute_ex(0, NK - 1)
        scale_rows(0)
        wait_scatters(1)
        pltpu.sync_copy(ex0, den_sh.at[sd0.at[1]], add=True)
        pltpu.sync_copy(rows0, acc_sh.at[sd0.at[1]], add=True)

        plsc.subcore_barrier()

        pltpu.sync_copy(acc_sh.at[pl.ds(sid * SH, SH)],
                        acc_hbm.at[cid, pl.ds(sid * SH, SH)])
        pltpu.sync_copy(den_sh.at[pl.ds(sid * SH, SH)],
                        den_hbm.at[cid, pl.ds(sid * SH, SH)])

    return k(xs, asrc, adst, sd3)


def kernel(x, edge_index, edge_attr, W1, a_s1, a_d1, b1, W2, a_s2, a_d2, b2):
    ei = jnp.pad(edge_index.astype(jnp.int32), ((0, 0), (0, EPAD - E)))
    sd3 = jnp.stack([ei[0].reshape(NW, NK, CH), ei[1].reshape(NW, NK, CH)],
                    axis=2)

    xs1, as1, ad1 = _tc_project(x, W1, a_s1, a_d1)
    acc1, den1 = _sc_edge_layer(xs1, as1.reshape(N), ad1.reshape(N), sd3)
    xs2, as2, ad2 = _tc_combine_project(acc1, den1, b1, W2, a_s2, a_d2)
    acc2, den2 = _sc_edge_layer(xs2, as2.reshape(N), ad2.reshape(N), sd3)
    return _tc_combine_final(acc2, den2, b2)


# speedup vs baseline: 1.4837x; 1.0017x over previous
import dataclasses
import functools
import jax
import jax.numpy as jnp
from jax import lax
from jax.experimental import pallas as pl
from jax.experimental.pallas import tpu as pltpu
from jax.experimental.pallas import tpu_sc as plsc

N = 10000
D = 128
E = 320000
NC = 2
NS = 16
NW = NC * NS
EW = E // NW
CH = 80
NK = -(-EW // CH)
EWP = NK * CH
EPAD = NW * EWP
NPAD = 10240
SH = NPAD // NS
R = 2000


def _tc_project(x, W, a_s, a_d):
    def body(x_ref, w_ref, as_ref, ad_ref, xs_ref, als_ref, ald_ref):
        xs = jnp.dot(x_ref[...], w_ref[...], preferred_element_type=jnp.float32)
        xs_ref[...] = xs
        als_ref[...] = jnp.dot(xs, as_ref[...], preferred_element_type=jnp.float32)
        ald_ref[...] = jnp.dot(xs, ad_ref[...], preferred_element_type=jnp.float32)
    return pl.pallas_call(
        body,
        grid=(N // R,),
        in_specs=[pl.BlockSpec((R, D), lambda i: (i, 0)),
                  pl.BlockSpec((D, D), lambda i: (0, 0)),
                  pl.BlockSpec((D, 1), lambda i: (0, 0)),
                  pl.BlockSpec((D, 1), lambda i: (0, 0))],
        out_specs=[pl.BlockSpec((R, D), lambda i: (i, 0)),
                   pl.BlockSpec((R, 1), lambda i: (i, 0)),
                   pl.BlockSpec((R, 1), lambda i: (i, 0))],
        out_shape=[jax.ShapeDtypeStruct((N, D), jnp.float32),
                   jax.ShapeDtypeStruct((N, 1), jnp.float32),
                   jax.ShapeDtypeStruct((N, 1), jnp.float32)],
    )(x, W, a_s.reshape(D, 1), a_d.reshape(D, 1))


def _tc_combine_project(acc, den, b, W, a_s, a_d):
    def body(acc_ref, den_ref, b_ref, w_ref, as_ref, ad_ref,
             xs_ref, als_ref, ald_ref):
        comb = acc_ref[0] + acc_ref[1]
        dd = den_ref[0] + den_ref[1] + 1e-16
        h = jnp.maximum(comb / dd + b_ref[...], 0.0)
        xs = jnp.dot(h, w_ref[...], preferred_element_type=jnp.float32)
        xs_ref[...] = xs
        als_ref[...] = jnp.dot(xs, as_ref[...], preferred_element_type=jnp.float32)
        ald_ref[...] = jnp.dot(xs, ad_ref[...], preferred_element_type=jnp.float32)
    return pl.pallas_call(
        body,
        grid=(N // R,),
        in_specs=[pl.BlockSpec((2, R, D), lambda i: (0, i, 0)),
                  pl.BlockSpec((2, R, 1), lambda i: (0, i, 0)),
                  pl.BlockSpec((1, D), lambda i: (0, 0)),
                  pl.BlockSpec((D, D), lambda i: (0, 0)),
                  pl.BlockSpec((D, 1), lambda i: (0, 0)),
                  pl.BlockSpec((D, 1), lambda i: (0, 0))],
        out_specs=[pl.BlockSpec((R, D), lambda i: (i, 0)),
                   pl.BlockSpec((R, 1), lambda i: (i, 0)),
                   pl.BlockSpec((R, 1), lambda i: (i, 0))],
        out_shape=[jax.ShapeDtypeStruct((N, D), jnp.float32),
                   jax.ShapeDtypeStruct((N, 1), jnp.float32),
                   jax.ShapeDtypeStruct((N, 1), jnp.float32)],
    )(acc, den.reshape(NC, NPAD, 1), b.reshape(1, D),
      W, a_s.reshape(D, 1), a_d.reshape(D, 1))


def _tc_combine_final(acc, den, b):
    def body(acc_ref, den_ref, b_ref, out_ref):
        comb = acc_ref[0] + acc_ref[1]
        dd = den_ref[0] + den_ref[1] + 1e-16
        out_ref[...] = comb / dd + b_ref[...]
    return pl.pallas_call(
        body,
        grid=(N // R,),
        in_specs=[pl.BlockSpec((2, R, D), lambda i: (0, i, 0)),
                  pl.BlockSpec((2, R, 1), lambda i: (0, i, 0)),
                  pl.BlockSpec((1, D), lambda i: (0, 0))],
        out_specs=pl.BlockSpec((R, D), lambda i: (i, 0)),
        out_shape=jax.ShapeDtypeStruct((N, D), jnp.float32),
    )(acc, den.reshape(NC, NPAD, 1), b.reshape(1, D))


def _sc_edge_layer(xs, asrc, adst, sd3):
    mesh = plsc.VectorSubcoreMesh(core_axis_name="c", subcore_axis_name="s")
    cp = pltpu.CompilerParams()
    if "needs_layout_passes" in pltpu.CompilerParams.__dataclass_fields__:
        cp = dataclasses.replace(cp, needs_layout_passes=False)

    @functools.partial(
        pl.kernel,
        compiler_params=cp,
        out_type=[jax.ShapeDtypeStruct((NC, NPAD, D), jnp.float32),
                  jax.ShapeDtypeStruct((NC, NPAD), jnp.float32)],
        mesh=mesh,
        scratch_types=[
            pltpu.VMEM((2, CH), jnp.int32),
            pltpu.VMEM((2, CH), jnp.int32),
            pltpu.VMEM((CH,), jnp.float32),
            pltpu.VMEM((CH,), jnp.float32),
            pltpu.VMEM((CH,), jnp.float32),
            pltpu.VMEM((CH,), jnp.float32),
            pltpu.VMEM((CH,), jnp.float32),
            pltpu.VMEM((CH,), jnp.float32),
            pltpu.VMEM((CH, D), jnp.float32),
            pltpu.VMEM((CH, D), jnp.float32),
            pltpu.VMEM((SH,), jnp.float32),
            pltpu.VMEM_SHARED((NPAD, D), jnp.float32),
            pltpu.VMEM_SHARED((NPAD,), jnp.float32),
            pltpu.SemaphoreType.DMA,
            pltpu.SemaphoreType.DMA,
            pltpu.SemaphoreType.DMA,
            pltpu.SemaphoreType.DMA,
            pltpu.SemaphoreType.DMA,
            pltpu.SemaphoreType.DMA,
        ],
    )
    def k(xs_hbm, asrc_hbm, adst_hbm, sd_hbm,
          acc_hbm, den_hbm,
          sd0, sd1, a10, a11, a20, a21, ex0, ex1,
          rows0, rows1, zden_v, acc_sh, den_sh,
          isem0, isem1, gsem0, gsem1, ssem0, ssem1):
        cid = lax.axis_index("c")
        sid = lax.axis_index("s")
        wid = cid * NS + sid
        z16 = jnp.zeros((16,), jnp.float32)
        sdb = (sd0, sd1)
        a1b, a2b = (a10, a11), (a20, a21)
        exb, rowsb = (ex0, ex1), (rows0, rows1)
        isem, gsem, ssem = (isem0, isem1), (gsem0, gsem1), (ssem0, ssem1)

        @pl.loop(0, CH)
        def _(r):
            for c in range(D // 16):
                rows0[r, pl.ds(c * 16, 16)] = z16

        @pl.loop(0, SH, step=16)
        def _(i):
            zden_v[pl.ds(i, 16)] = z16

        for i in range(SH // CH):
            pltpu.sync_copy(rows0, acc_sh.at[pl.ds(sid * SH + i * CH, CH)])
        pltpu.sync_copy(zden_v, den_sh.at[pl.ds(sid * SH, SH)])

        plsc.subcore_barrier()

        def fire_idx(b, kk):
            pltpu.async_copy(sd_hbm.at[wid, kk], sdb[b], isem[b])

        def wait_idx(b, kk):
            pltpu.make_async_copy(sd_hbm.at[wid, kk], sdb[b], isem[b]).wait()

        def fire_gathers(b):
            pltpu.async_copy(asrc_hbm.at[sdb[b].at[0]], a1b[b], gsem[b])
            pltpu.async_copy(adst_hbm.at[sdb[b].at[1]], a2b[b], gsem[b])
            pltpu.async_copy(xs_hbm.at[sdb[b].at[0]], rowsb[b], gsem[b])

        def wait_gathers(b):
            pltpu.make_async_copy(asrc_hbm.at[sdb[b].at[0]], a1b[b], gsem[b]).wait()
            pltpu.make_async_copy(adst_hbm.at[sdb[b].at[1]], a2b[b], gsem[b]).wait()
            pltpu.make_async_copy(xs_hbm.at[sdb[b].at[0]], rowsb[b], gsem[b]).wait()

        def fire_scatters(b):
            pltpu.async_copy(exb[b], den_sh.at[sdb[b].at[1]], ssem[b], add=True)
            pltpu.async_copy(rowsb[b], acc_sh.at[sdb[b].at[1]], ssem[b], add=True)

        def wait_scatters(b):
            pltpu.make_async_copy(exb[b], den_sh.at[sdb[b].at[1]], ssem[b]).wait()
            pltpu.make_async_copy(rowsb[b], acc_sh.at[sdb[b].at[1]], ssem[b]).wait()

        def compute_ex(b, kk):
            if EPAD == E:
                @pl.loop(0, CH, step=16)
                def _(j):
                    e = a1b[b][pl.ds(j, 16)] + a2b[b][pl.ds(j, 16)]
                    e = jnp.maximum(e, 0.2 * e)
                    exb[b][pl.ds(j, 16)] = jnp.exp(e)
                return
            base = wid * EWP + kk * CH
            lanes = lax.iota(jnp.int32, 16)

            @pl.loop(0, CH, step=16)
            def _(j):
                e = a1b[b][pl.ds(j, 16)] + a2b[b][pl.ds(j, 16)]
                e = jnp.maximum(e, 0.2 * e)
                ok = (base + j + lanes) < E
                exb[b][pl.ds(j, 16)] = jnp.where(ok, jnp.exp(e), 0.0)

        def scale_rows(b):
            @plsc.parallel_loop(0, CH, step=16, unroll=2)
            def _(j):
                ex16 = exb[b][pl.ds(j, 16)]
                for rr in range(16):
                    sv = ex16[rr]
                    for c in range(D // 16):
                        sl = pl.ds(c * 16, 16)
                        rowsb[b][j + rr, sl] = rowsb[b][j + rr, sl] * sv

        fire_idx(0, 0)
        wait_idx(0, 0)
        fire_gathers(0)

        @pl.loop(0, NK - 1, step=2)
        def _(kk):
            @pl.when(kk > 0)
            def _():
                wait_scatters(1)
            fire_idx(1, kk + 1)
            wait_gathers(0)
            compute_ex(0, kk)
            wait_idx(1, kk + 1)
            fire_gathers(1)
            scale_rows(0)
            fire_scatters(0)
            wait_gathers(1)
            compute_ex(1, kk + 1)
            wait_scatters(0)
            fire_idx(0, kk + 2)
            wait_idx(0, kk + 2)
            fire_gathers(0)
            scale_rows(1)
            fire_scatters(1)

        wait_gathers(0)
        compute_ex(0, NK - 1)
        scale_rows(0)
        wait_scatters(1)
        pltpu.sync_copy(ex0, den_sh.at[sd0.at[1]], add=True)
        pltpu.sync_copy(rows0, acc_sh.at[sd0.at[1]], add=True)

        plsc.subcore_barrier()

        pltpu.sync_copy(acc_sh.at[pl.ds(sid * SH, SH)],
                        acc_hbm.at[cid, pl.ds(sid * SH, SH)])
        pltpu.sync_copy(den_sh.at[pl.ds(sid * SH, SH)],
                        den_hbm.at[cid, pl.ds(sid * SH, SH)])

    return k(xs, asrc, adst, sd3)


def kernel(x, edge_index, edge_attr, W1, a_s1, a_d1, b1, W2, a_s2, a_d2, b2):
    ei = jnp.pad(edge_index.astype(jnp.int32), ((0, 0), (0, EPAD - E)))
    sd3 = jnp.stack([ei[0].reshape(NW, NK, CH), ei[1].reshape(NW, NK, CH)],
                    axis=2)

    xs1, as1, ad1 = _tc_project(x, W1, a_s1, a_d1)
    acc1, den1 = _sc_edge_layer(xs1, as1.reshape(N), ad1.reshape(N), sd3)
    xs2, as2, ad2 = _tc_combine_project(acc1, den1, b1, W2, a_s2, a_d2)
    acc2, den2 = _sc_edge_layer(xs2, as2.reshape(N), ad2.reshape(N), sd3)
    return _tc_combine_final(acc2, den2, b2)
